# Initial kernel scaffold; baseline (speedup 1.0000x reference)
import jax
import jax.numpy as jnp
from jax.experimental import pallas as pl


def kernel(x, edge_index, edge_weight, g_size, W_emb, b_emb, wc1_W1, wc1_b1, wc1_W2, wc1_b2, gc1_W, gc1_b, wc2_W1, wc2_b1, wc2_W2, wc2_b2, gc2_W, gc2_b, wcl_W1, wcl_b1, wcl_W2, wcl_b2, fc_W, fc_b):
    raise NotImplementedError("write your pallas kernel here")


# speedup vs baseline: 1.2029x; 1.2029x over previous
import functools

import jax
import jax.numpy as jnp
from jax import lax
from jax.experimental import pallas as pl
from jax.experimental.pallas import tpu as pltpu
from jax.experimental.pallas import tpu_sc as plsc

N_NODES = 10000
N_EDGES = 160000
EP = 163840
HID = 256
NP = 10240
BN = 512
BE = 640
NSC = 2
NTILES = 16
NW = NSC * NTILES
EPW = EP // NW
KA = 128
EPT = EP // NTILES
KS = 128
ZB = 64

@functools.cache
def _sc_mesh():
    return plsc.VectorSubcoreMesh(core_axis_name="c", subcore_axis_name="s",
                                  num_cores=NSC, num_subcores=NTILES)


def _tc_matmul(ins, weights, bias, relu):
    rows = ins[0].shape[0]
    out_dim = weights[0].shape[1]
    n_in = len(ins)
    grid = (rows // BN,)

    def body(*refs):
        in_refs = refs[:n_in]
        w_refs = refs[n_in:2 * n_in]
        b_ref = refs[2 * n_in]
        out_ref = refs[2 * n_in + 1]
        acc = jnp.dot(in_refs[0][...], w_refs[0][...],
                      preferred_element_type=jnp.float32)
        for k in range(1, n_in):
            acc = acc + jnp.dot(in_refs[k][...], w_refs[k][...],
                                preferred_element_type=jnp.float32)
        acc = acc + b_ref[...]
        if relu:
            acc = jnp.maximum(acc, 0.0)
        out_ref[...] = acc

    in_specs = (
        [pl.BlockSpec((BN, a.shape[1]), lambda i: (i, 0)) for a in ins]
        + [pl.BlockSpec(w.shape, lambda i: (0, 0)) for w in weights]
        + [pl.BlockSpec((1, out_dim), lambda i: (0, 0))]
    )
    return pl.pallas_call(
        body,
        grid=grid,
        in_specs=in_specs,
        out_specs=pl.BlockSpec((BN, out_dim), lambda i: (i, 0)),
        out_shape=jax.ShapeDtypeStruct((rows, out_dim), jnp.float32),
    )(*ins, *weights, bias.reshape(1, out_dim))


def _tc_edge_mlp(d_blocks, w1_blocks, b1, w2_row, b2, edge_weight):
    n_in = len(d_blocks)
    grid = (EP // BE,)

    def body(*refs):
        d_refs = refs[:n_in]
        w_refs = refs[n_in:2 * n_in]
        b1_ref, w2_ref, b2_ref, ew_ref, out_ref = refs[2 * n_in:]
        acc = jnp.dot(d_refs[0][...], w_refs[0][...],
                      preferred_element_type=jnp.float32)
        for k in range(1, n_in):
            acc = acc + jnp.dot(d_refs[k][...], w_refs[k][...],
                                preferred_element_type=jnp.float32)
        h1 = jnp.maximum(acc + b1_ref[...], 0.0)
        t = jnp.sum(h1 * w2_ref[...], axis=1) + b2_ref[...][0, 0]
        w = 1.0 / (1.0 + jnp.exp(-t))
        out_ref[0, 0, :] = w * ew_ref[0, 0, :]

    in_specs = (
        [pl.BlockSpec((BE, HID), lambda i: (i, 0)) for _ in d_blocks]
        + [pl.BlockSpec((HID, HID), lambda i: (0, 0)) for _ in w1_blocks]
        + [pl.BlockSpec((1, HID), lambda i: (0, 0)),
           pl.BlockSpec((1, HID), lambda i: (0, 0)),
           pl.BlockSpec((1, 1), lambda i: (0, 0)),
           pl.BlockSpec((1, 1, BE), lambda i: (i, 0, 0))]
    )
    return pl.pallas_call(
        body,
        grid=grid,
        in_specs=in_specs,
        out_specs=pl.BlockSpec((1, 1, BE), lambda i: (i, 0, 0)),
        out_shape=jax.ShapeDtypeStruct((EP // BE, 1, BE), jnp.float32),
    )(*d_blocks, *w1_blocks, b1.reshape(1, HID), w2_row.reshape(1, HID),
      b2.reshape(1, 1), edge_weight.reshape(EP // BE, 1, BE)
      ).reshape(EP)


@functools.cache
def _sc_absdiff_kernel():
    return pl.kernel(
        _sc_absdiff_body,
        out_type=jax.ShapeDtypeStruct((EP, HID), jnp.float32),
        mesh=_sc_mesh(),
        scratch_types=[
            pltpu.VMEM((KA,), jnp.int32),
            pltpu.VMEM((KA,), jnp.int32),
            pltpu.VMEM((KA, HID), jnp.float32),
            pltpu.VMEM((KA, HID), jnp.float32),
            pltpu.SemaphoreType.DMA,
            pltpu.SemaphoreType.DMA,
        ],
    )


def _sc_absdiff_body(table, src, dst, out, si, di, hs, hd, sem_s, sem_d):
    wid = lax.axis_index("s") * NSC + lax.axis_index("c")
    base = wid * EPW

    def chunk(c, carry):
        off = base + c * KA
        pltpu.sync_copy(src.at[pl.ds(off, KA)], si)
        pltpu.sync_copy(dst.at[pl.ds(off, KA)], di)
        cp_s = pltpu.async_copy(table.at[si], hs, sem_s)
        cp_d = pltpu.async_copy(table.at[di], hd, sem_d)
        cp_s.wait()
        cp_d.wait()

        def row(r, carry2):
            for g in range(HID // 16):
                sl = pl.ds(g * 16, 16)
                hs[r, sl] = jnp.abs(hs[r, sl] - hd[r, sl])
            return carry2

        lax.fori_loop(0, KA, row, 0)
        pltpu.sync_copy(hs, out.at[pl.ds(off, KA)])
        return carry

    lax.fori_loop(0, EPW // KA, chunk, 0)


@functools.cache
def _sc_scatter_kernel():
    return pl.kernel(
        _sc_scatter_body,
        out_type=jax.ShapeDtypeStruct((2 * NP, 128), jnp.float32),
        mesh=_sc_mesh(),
        scratch_types=[
            pltpu.VMEM((KS,), jnp.int32),
            pltpu.VMEM((KS,), jnp.int32),
            pltpu.VMEM((KS,), jnp.float32),
            pltpu.VMEM((KS, 128), jnp.float32),
            pltpu.VMEM((ZB, 128), jnp.float32),
            pltpu.VMEM_SHARED((NP, 128), jnp.float32),
            pltpu.SemaphoreType.DMA,
        ],
    )


def _sc_scatter_body(table2, src, dst, w, out, si, di, wv, rows, zbuf, agg, sem):
    cid = lax.axis_index("c")
    sid = lax.axis_index("s")

    def zrow(r, carry):
        for g in range(8):
            zbuf[r, pl.ds(g * 16, 16)] = jnp.zeros((16,), jnp.float32)
        return carry

    lax.fori_loop(0, ZB, zrow, 0)
    rows_per_tile = NP // NTILES
    for z in range(rows_per_tile // ZB):
        pltpu.sync_copy(zbuf, agg.at[pl.ds(sid * rows_per_tile + z * ZB, ZB)])
    plsc.subcore_barrier()

    base = sid * EPT
    col_off = cid * NP

    def chunk(c, carry):
        off = base + c * KS
        pltpu.sync_copy(src.at[pl.ds(off, KS)], si)
        pltpu.sync_copy(dst.at[pl.ds(off, KS)], di)
        pltpu.sync_copy(w.at[pl.ds(off, KS)], wv)
        def adj(g, carry2):
            sl = pl.ds(g * 16, 16)
            si[sl] = si[sl] + col_off
            return carry2

        lax.fori_loop(0, KS // 16, adj, 0)
        pltpu.async_copy(table2.at[si], rows, sem).wait()

        def scale(q, carry2):
            w16 = wv[pl.ds(q * 16, 16)]
            for j in range(16):
                r = q * 16 + j
                ws = w16[j]
                for g in range(8):
                    sl = pl.ds(g * 16, 16)
                    rows[r, sl] = rows[r, sl] * ws
            return carry2

        lax.fori_loop(0, KS // 16, scale, 0)
        pltpu.sync_copy(rows, agg.at[di], add=True)
        return carry

    lax.fori_loop(0, EPT // KS, chunk, 0)
    plsc.subcore_barrier()
    pltpu.sync_copy(
        agg.at[pl.ds(sid * rows_per_tile, rows_per_tile)],
        out.at[pl.ds(col_off + sid * rows_per_tile, rows_per_tile)],
    )


def kernel(x, edge_index, edge_weight, g_size, W_emb, b_emb,
           wc1_W1, wc1_b1, wc1_W2, wc1_b2, gc1_W, gc1_b,
           wc2_W1, wc2_b1, wc2_W2, wc2_b2, gc2_W, gc2_b,
           wcl_W1, wcl_b1, wcl_W2, wcl_b2, fc_W, fc_b):
    src = jnp.pad(edge_index[0], (0, EP - N_EDGES))
    dst = jnp.pad(edge_index[1], (0, EP - N_EDGES))
    edge_weight = jnp.pad(edge_weight, (0, EP - N_EDGES))

    x_p = jnp.pad(x, ((0, NP - N_NODES), (0, 0)))
    h0 = _tc_matmul([x_p], [W_emb], b_emb, relu=True)

    def chunks(h):
        return jnp.concatenate([h[:, :128], h[:, 128:]], axis=0)

    d0 = _sc_absdiff_kernel()(h0, src, dst)
    w1 = _tc_edge_mlp([d0], [wc1_W1], wc1_b1, wc1_W2[:, 0], wc1_b2,
                      edge_weight)
    agg1 = _sc_scatter_kernel()(chunks(h0), src, dst, w1)
    hn1 = _tc_matmul(
        [h0, agg1[:NP], agg1[NP:]],
        [gc1_W[0:256], gc1_W[256:384], gc1_W[384:512]],
        gc1_b, relu=True)

    d1 = _sc_absdiff_kernel()(hn1, src, dst)
    w2 = _tc_edge_mlp([d0, d1], [wc2_W1[0:256], wc2_W1[256:512]],
                      wc2_b1, wc2_W2[:, 0], wc2_b2, edge_weight)
    agg2a = _sc_scatter_kernel()(chunks(h0), src, dst, w2)
    agg2b = _sc_scatter_kernel()(chunks(hn1), src, dst, w2)
    hn2 = _tc_matmul(
        [h0, hn1, agg2a[:NP], agg2a[NP:], agg2b[:NP], agg2b[NP:]],
        [gc2_W[0:256], gc2_W[256:512], gc2_W[512:640], gc2_W[640:768],
         gc2_W[768:896], gc2_W[896:1024]],
        gc2_b, relu=True)

    d2 = _sc_absdiff_kernel()(hn2, src, dst)
    wl = _tc_edge_mlp([d0, d1, d2],
                      [wcl_W1[0:256], wcl_W1[256:512], wcl_W1[512:768]],
                      wcl_b1, wcl_W2[:, 0], wcl_b2, edge_weight)[:N_EDGES]
    out = _tc_matmul([h0, hn1, hn2],
                     [fc_W[0:256], fc_W[256:512], fc_W[512:768]],
                     fc_b, relu=False)[:N_NODES]
    return (out, wl, g_size)


# speedup vs baseline: 1.4812x; 1.2313x over previous
import functools

import jax
import jax.numpy as jnp
from jax import lax
from jax.experimental import pallas as pl
from jax.experimental.pallas import tpu as pltpu
from jax.experimental.pallas import tpu_sc as plsc

N_NODES = 10000
N_EDGES = 160000
EP = 163840
HID = 256
NP = 10240
BN = 512
BE = 640
NSC = 2
NTILES = 16
NW = NSC * NTILES
EPW = EP // NW
KA = 64
EPT = EP // NTILES
KS = 128
ZB = 64

@functools.cache
def _sc_mesh():
    return plsc.VectorSubcoreMesh(core_axis_name="c", subcore_axis_name="s",
                                  num_cores=NSC, num_subcores=NTILES)


def _tc_matmul(ins, weights, bias, relu):
    rows = ins[0].shape[0]
    out_dim = weights[0].shape[1]
    n_in = len(ins)
    grid = (rows // BN,)

    def body(*refs):
        in_refs = refs[:n_in]
        w_refs = refs[n_in:2 * n_in]
        b_ref = refs[2 * n_in]
        out_ref = refs[2 * n_in + 1]
        acc = jnp.dot(in_refs[0][...], w_refs[0][...],
                      preferred_element_type=jnp.float32)
        for k in range(1, n_in):
            acc = acc + jnp.dot(in_refs[k][...], w_refs[k][...],
                                preferred_element_type=jnp.float32)
        acc = acc + b_ref[...]
        if relu:
            acc = jnp.maximum(acc, 0.0)
        out_ref[...] = acc

    in_specs = (
        [pl.BlockSpec((BN, a.shape[1]), lambda i: (i, 0)) for a in ins]
        + [pl.BlockSpec(w.shape, lambda i: (0, 0)) for w in weights]
        + [pl.BlockSpec((1, out_dim), lambda i: (0, 0))]
    )
    return pl.pallas_call(
        body,
        grid=grid,
        in_specs=in_specs,
        out_specs=pl.BlockSpec((BN, out_dim), lambda i: (i, 0)),
        out_shape=jax.ShapeDtypeStruct((rows, out_dim), jnp.float32),
    )(*ins, *weights, bias.reshape(1, out_dim))


def _tc_edge_mlp(d_blocks, w1_blocks, b1, w2_row, b2, edge_weight):
    n_in = len(d_blocks)
    grid = (EP // BE,)

    def body(*refs):
        d_refs = refs[:n_in]
        w_refs = refs[n_in:2 * n_in]
        b1_ref, w2_ref, b2_ref, ew_ref, out_ref = refs[2 * n_in:]
        acc = jnp.dot(d_refs[0][...], w_refs[0][...],
                      preferred_element_type=jnp.float32)
        for k in range(1, n_in):
            acc = acc + jnp.dot(d_refs[k][...], w_refs[k][...],
                                preferred_element_type=jnp.float32)
        h1 = jnp.maximum(acc + b1_ref[...], 0.0)
        t = jnp.sum(h1 * w2_ref[...], axis=1) + b2_ref[...][0, 0]
        w = 1.0 / (1.0 + jnp.exp(-t))
        out_ref[0, 0, :] = w * ew_ref[0, 0, :]

    in_specs = (
        [pl.BlockSpec((BE, HID), lambda i: (i, 0)) for _ in d_blocks]
        + [pl.BlockSpec((HID, HID), lambda i: (0, 0)) for _ in w1_blocks]
        + [pl.BlockSpec((1, HID), lambda i: (0, 0)),
           pl.BlockSpec((1, HID), lambda i: (0, 0)),
           pl.BlockSpec((1, 1), lambda i: (0, 0)),
           pl.BlockSpec((1, 1, BE), lambda i: (i, 0, 0))]
    )
    return pl.pallas_call(
        body,
        grid=grid,
        in_specs=in_specs,
        out_specs=pl.BlockSpec((1, 1, BE), lambda i: (i, 0, 0)),
        out_shape=jax.ShapeDtypeStruct((EP // BE, 1, BE), jnp.float32),
    )(*d_blocks, *w1_blocks, b1.reshape(1, HID), w2_row.reshape(1, HID),
      b2.reshape(1, 1), edge_weight.reshape(EP // BE, 1, BE)
      ).reshape(EP)


@functools.cache
def _sc_absdiff_kernel():
    return pl.kernel(
        _sc_absdiff_body,
        out_type=jax.ShapeDtypeStruct((EP, HID), jnp.float32),
        mesh=_sc_mesh(),
        scratch_types=[
            pltpu.VMEM((2, KA), jnp.int32),
            pltpu.VMEM((2, KA), jnp.int32),
            pltpu.VMEM((KA, HID), jnp.float32),
            pltpu.VMEM((KA, HID), jnp.float32),
            pltpu.VMEM((KA, HID), jnp.float32),
            pltpu.VMEM((KA, HID), jnp.float32),
            pltpu.VMEM((KA, HID), jnp.float32),
            pltpu.VMEM((KA, HID), jnp.float32),
            pltpu.SemaphoreType.DMA,
            pltpu.SemaphoreType.DMA,
            pltpu.SemaphoreType.DMA,
            pltpu.SemaphoreType.DMA,
        ],
    )


def _sc_absdiff_body(table, src, dst, out,
                     si, di, hs0, hd0, hs1, hd1, ob0, ob1,
                     sg0, sg1, sw0, sw1):
    wid = lax.axis_index("s") * NSC + lax.axis_index("c")
    base = wid * EPW
    nch = EPW // KA
    hsb, hdb, obb = (hs0, hs1), (hd0, hd1), (ob0, ob1)
    sgb, swb = (sg0, sg1), (sw0, sw1)

    def issue(c, b):
        off = base + c * KA
        pltpu.sync_copy(src.at[pl.ds(off, KA)], si.at[b])
        pltpu.sync_copy(dst.at[pl.ds(off, KA)], di.at[b])
        pltpu.async_copy(table.at[si.at[b]], hsb[b], sgb[b])
        pltpu.async_copy(table.at[di.at[b]], hdb[b], sgb[b])

    issue(0, 0)
    issue(1, 1)

    def pair(cc, carry):
        for b in range(2):
            c = cc * 2 + b
            pltpu.make_async_copy(table.at[si.at[b]], hsb[b], sgb[b]).wait()
            pltpu.make_async_copy(table.at[di.at[b]], hdb[b], sgb[b]).wait()

            @pl.when(cc > 0)
            def _():
                pltpu.make_async_copy(
                    obb[b], out.at[pl.ds(0, KA)], swb[b]).wait()

            def row(r, carry2):
                for g in range(HID // 16):
                    sl = pl.ds(g * 16, 16)
                    obb[b][r, sl] = jnp.abs(hsb[b][r, sl] - hdb[b][r, sl])
                return carry2

            lax.fori_loop(0, KA, row, 0)
            pltpu.async_copy(obb[b], out.at[pl.ds(base + c * KA, KA)], swb[b])

            @pl.when(cc < nch // 2 - 1)
            def _():
                issue(c + 2, b)
        return carry

    lax.fori_loop(0, nch // 2, pair, 0)
    pltpu.make_async_copy(ob0, out.at[pl.ds(0, KA)], sw0).wait()
    pltpu.make_async_copy(ob1, out.at[pl.ds(0, KA)], sw1).wait()


@functools.cache
def _sc_scatter_kernel():
    return pl.kernel(
        _sc_scatter_body,
        out_type=jax.ShapeDtypeStruct((2 * NP, 128), jnp.float32),
        mesh=_sc_mesh(),
        scratch_types=[
            pltpu.VMEM((2, KS), jnp.int32),
            pltpu.VMEM((2, KS), jnp.int32),
            pltpu.VMEM((2, KS), jnp.float32),
            pltpu.VMEM((KS, 128), jnp.float32),
            pltpu.VMEM((KS, 128), jnp.float32),
            pltpu.VMEM((ZB, 128), jnp.float32),
            pltpu.VMEM_SHARED((NP, 128), jnp.float32),
            pltpu.SemaphoreType.DMA,
            pltpu.SemaphoreType.DMA,
        ],
    )


def _sc_scatter_body(table2, src, dst, w, out, si, di, wv, rows0, rows1,
                     zbuf, agg, sg0, sg1):
    cid = lax.axis_index("c")
    sid = lax.axis_index("s")
    rowsb = (rows0, rows1)
    sgb = (sg0, sg1)

    def zrow(r, carry):
        for g in range(8):
            zbuf[r, pl.ds(g * 16, 16)] = jnp.zeros((16,), jnp.float32)
        return carry

    lax.fori_loop(0, ZB, zrow, 0)
    rows_per_tile = NP // NTILES
    for z in range(rows_per_tile // ZB):
        pltpu.sync_copy(zbuf, agg.at[pl.ds(sid * rows_per_tile + z * ZB, ZB)])
    plsc.subcore_barrier()

    base = sid * EPT
    col_off = cid * NP
    nch = EPT // KS

    def issue(c, b):
        off = base + c * KS
        pltpu.sync_copy(src.at[pl.ds(off, KS)], si.at[b])
        pltpu.sync_copy(dst.at[pl.ds(off, KS)], di.at[b])
        pltpu.sync_copy(w.at[pl.ds(off, KS)], wv.at[b])
        def adj(g, carry2):
            sl = pl.ds(g * 16, 16)
            si[b, sl] = si[b, sl] + col_off
            return carry2

        lax.fori_loop(0, KS // 16, adj, 0)
        pltpu.async_copy(table2.at[si.at[b]], rowsb[b], sgb[b])

    issue(0, 0)
    issue(1, 1)

    def pair(cc, carry):
        for b in range(2):
            c = cc * 2 + b
            pltpu.make_async_copy(table2.at[si.at[b]], rowsb[b], sgb[b]).wait()

            def scale(q, carry2):
                w16 = wv[b, pl.ds(q * 16, 16)]
                for j in range(16):
                    r = q * 16 + j
                    ws = w16[j]
                    for g in range(8):
                        sl = pl.ds(g * 16, 16)
                        rowsb[b][r, sl] = rowsb[b][r, sl] * ws
                return carry2

            lax.fori_loop(0, KS // 16, scale, 0)
            pltpu.sync_copy(rowsb[b], agg.at[di.at[b]], add=True)

            @pl.when(cc < nch // 2 - 1)
            def _():
                issue(c + 2, b)
        return carry

    lax.fori_loop(0, nch // 2, pair, 0)
    plsc.subcore_barrier()
    pltpu.sync_copy(
        agg.at[pl.ds(sid * rows_per_tile, rows_per_tile)],
        out.at[pl.ds(col_off + sid * rows_per_tile, rows_per_tile)],
    )


def kernel(x, edge_index, edge_weight, g_size, W_emb, b_emb,
           wc1_W1, wc1_b1, wc1_W2, wc1_b2, gc1_W, gc1_b,
           wc2_W1, wc2_b1, wc2_W2, wc2_b2, gc2_W, gc2_b,
           wcl_W1, wcl_b1, wcl_W2, wcl_b2, fc_W, fc_b):
    src = jnp.pad(edge_index[0], (0, EP - N_EDGES))
    dst = jnp.pad(edge_index[1], (0, EP - N_EDGES))
    edge_weight = jnp.pad(edge_weight, (0, EP - N_EDGES))

    x_p = jnp.pad(x, ((0, NP - N_NODES), (0, 0)))
    h0 = _tc_matmul([x_p], [W_emb], b_emb, relu=True)

    def chunks(h):
        return jnp.concatenate([h[:, :128], h[:, 128:]], axis=0)

    d0 = _sc_absdiff_kernel()(h0, src, dst)
    w1 = _tc_edge_mlp([d0], [wc1_W1], wc1_b1, wc1_W2[:, 0], wc1_b2,
                      edge_weight)
    agg1 = _sc_scatter_kernel()(chunks(h0), src, dst, w1)
    hn1 = _tc_matmul(
        [h0, agg1[:NP], agg1[NP:]],
        [gc1_W[0:256], gc1_W[256:384], gc1_W[384:512]],
        gc1_b, relu=True)

    d1 = _sc_absdiff_kernel()(hn1, src, dst)
    w2 = _tc_edge_mlp([d0, d1], [wc2_W1[0:256], wc2_W1[256:512]],
                      wc2_b1, wc2_W2[:, 0], wc2_b2, edge_weight)
    agg2a = _sc_scatter_kernel()(chunks(h0), src, dst, w2)
    agg2b = _sc_scatter_kernel()(chunks(hn1), src, dst, w2)
    hn2 = _tc_matmul(
        [h0, hn1, agg2a[:NP], agg2a[NP:], agg2b[:NP], agg2b[NP:]],
        [gc2_W[0:256], gc2_W[256:512], gc2_W[512:640], gc2_W[640:768],
         gc2_W[768:896], gc2_W[896:1024]],
        gc2_b, relu=True)

    d2 = _sc_absdiff_kernel()(hn2, src, dst)
    wl = _tc_edge_mlp([d0, d1, d2],
                      [wcl_W1[0:256], wcl_W1[256:512], wcl_W1[512:768]],
                      wcl_b1, wcl_W2[:, 0], wcl_b2, edge_weight)[:N_EDGES]
    out = _tc_matmul([h0, hn1, hn2],
                     [fc_W[0:256], fc_W[256:512], fc_W[512:768]],
                     fc_b, relu=False)[:N_NODES]
    return (out, wl, g_size)


# speedup vs baseline: 1.5589x; 1.0525x over previous
import functools

import jax
import jax.numpy as jnp
from jax import lax
from jax.experimental import pallas as pl
from jax.experimental.pallas import tpu as pltpu
from jax.experimental.pallas import tpu_sc as plsc

N_NODES = 10000
N_EDGES = 160000
EP = 163840
HID = 256
NP = 10240
BN = 512
BE = 640
NSC = 2
NTILES = 16
NW = NSC * NTILES
EPW = EP // NW
KA = 64
EPT = EP // NTILES
KS = 80
ZB = 32

@functools.cache
def _sc_mesh():
    return plsc.VectorSubcoreMesh(core_axis_name="c", subcore_axis_name="s",
                                  num_cores=NSC, num_subcores=NTILES)


def _tc_matmul(ins, weights, bias, relu):
    rows = ins[0].shape[0]
    out_dim = weights[0].shape[1]
    n_in = len(ins)
    grid = (rows // BN,)

    def body(*refs):
        in_refs = refs[:n_in]
        w_refs = refs[n_in:2 * n_in]
        b_ref = refs[2 * n_in]
        out_ref = refs[2 * n_in + 1]
        acc = jnp.dot(in_refs[0][...], w_refs[0][...],
                      preferred_element_type=jnp.float32)
        for k in range(1, n_in):
            acc = acc + jnp.dot(in_refs[k][...], w_refs[k][...],
                                preferred_element_type=jnp.float32)
        acc = acc + b_ref[...]
        if relu:
            acc = jnp.maximum(acc, 0.0)
        out_ref[...] = acc

    in_specs = (
        [pl.BlockSpec((BN, a.shape[1]), lambda i: (i, 0)) for a in ins]
        + [pl.BlockSpec(w.shape, lambda i: (0, 0)) for w in weights]
        + [pl.BlockSpec((1, out_dim), lambda i: (0, 0))]
    )
    return pl.pallas_call(
        body,
        grid=grid,
        in_specs=in_specs,
        out_specs=pl.BlockSpec((BN, out_dim), lambda i: (i, 0)),
        out_shape=jax.ShapeDtypeStruct((rows, out_dim), jnp.float32),
    )(*ins, *weights, bias.reshape(1, out_dim))


def _tc_edge_mlp(d_blocks, w1_blocks, b1, w2_row, b2, edge_weight):
    n_in = len(d_blocks)
    grid = (EP // BE,)

    def body(*refs):
        d_refs = refs[:n_in]
        w_refs = refs[n_in:2 * n_in]
        b1_ref, w2_ref, b2_ref, ew_ref, out_ref = refs[2 * n_in:]
        acc = jnp.dot(d_refs[0][...], w_refs[0][...],
                      preferred_element_type=jnp.float32)
        for k in range(1, n_in):
            acc = acc + jnp.dot(d_refs[k][...], w_refs[k][...],
                                preferred_element_type=jnp.float32)
        h1 = jnp.maximum(acc + b1_ref[...], 0.0)
        t = jnp.sum(h1 * w2_ref[...], axis=1) + b2_ref[...][0, 0]
        w = 1.0 / (1.0 + jnp.exp(-t))
        out_ref[0, 0, :] = w * ew_ref[0, 0, :]

    in_specs = (
        [pl.BlockSpec((BE, HID), lambda i: (i, 0)) for _ in d_blocks]
        + [pl.BlockSpec((HID, HID), lambda i: (0, 0)) for _ in w1_blocks]
        + [pl.BlockSpec((1, HID), lambda i: (0, 0)),
           pl.BlockSpec((1, HID), lambda i: (0, 0)),
           pl.BlockSpec((1, 1), lambda i: (0, 0)),
           pl.BlockSpec((1, 1, BE), lambda i: (i, 0, 0))]
    )
    return pl.pallas_call(
        body,
        grid=grid,
        in_specs=in_specs,
        out_specs=pl.BlockSpec((1, 1, BE), lambda i: (i, 0, 0)),
        out_shape=jax.ShapeDtypeStruct((EP // BE, 1, BE), jnp.float32),
    )(*d_blocks, *w1_blocks, b1.reshape(1, HID), w2_row.reshape(1, HID),
      b2.reshape(1, 1), edge_weight.reshape(EP // BE, 1, BE)
      ).reshape(EP)


@functools.cache
def _sc_absdiff_kernel():
    return pl.kernel(
        _sc_absdiff_body,
        out_type=jax.ShapeDtypeStruct((EP, HID), jnp.float32),
        mesh=_sc_mesh(),
        scratch_types=[
            pltpu.VMEM((EPW,), jnp.int32),
            pltpu.VMEM((EPW,), jnp.int32),
            pltpu.VMEM((KA, HID), jnp.float32),
            pltpu.VMEM((KA, HID), jnp.float32),
            pltpu.VMEM((KA, HID), jnp.float32),
            pltpu.VMEM((KA, HID), jnp.float32),
            pltpu.VMEM((KA, HID), jnp.float32),
            pltpu.VMEM((KA, HID), jnp.float32),
            pltpu.SemaphoreType.DMA,
            pltpu.SemaphoreType.DMA,
            pltpu.SemaphoreType.DMA,
            pltpu.SemaphoreType.DMA,
        ],
    )


def _sc_absdiff_body(table, src, dst, out,
                     si, di, hs0, hd0, hs1, hd1, ob0, ob1,
                     sg0, sg1, sw0, sw1):
    wid = lax.axis_index("s") * NSC + lax.axis_index("c")
    base = wid * EPW
    nch = EPW // KA
    hsb, hdb, obb = (hs0, hs1), (hd0, hd1), (ob0, ob1)
    sgb, swb = (sg0, sg1), (sw0, sw1)

    pltpu.sync_copy(src.at[pl.ds(base, EPW)], si)
    pltpu.sync_copy(dst.at[pl.ds(base, EPW)], di)

    def issue(c, b):
        pltpu.async_copy(table.at[si.at[pl.ds(c * KA, KA)]], hsb[b], sgb[b])
        pltpu.async_copy(table.at[di.at[pl.ds(c * KA, KA)]], hdb[b], sgb[b])

    issue(0, 0)
    issue(1, 1)

    def pair(cc, carry):
        for b in range(2):
            c = cc * 2 + b
            pltpu.make_async_copy(table.at[si.at[pl.ds(0, KA)]],
                                  hsb[b], sgb[b]).wait()
            pltpu.make_async_copy(table.at[di.at[pl.ds(0, KA)]],
                                  hdb[b], sgb[b]).wait()

            @pl.when(cc > 0)
            def _():
                pltpu.make_async_copy(
                    obb[b], out.at[pl.ds(0, KA)], swb[b]).wait()

            def row(r, carry2):
                for g in range(HID // 16):
                    sl = pl.ds(g * 16, 16)
                    obb[b][r, sl] = jnp.abs(hsb[b][r, sl] - hdb[b][r, sl])
                return carry2

            lax.fori_loop(0, KA, row, 0)
            pltpu.async_copy(obb[b], out.at[pl.ds(base + c * KA, KA)], swb[b])

            @pl.when(cc < nch // 2 - 1)
            def _():
                issue(c + 2, b)
        return carry

    lax.fori_loop(0, nch // 2, pair, 0)
    pltpu.make_async_copy(ob0, out.at[pl.ds(0, KA)], sw0).wait()
    pltpu.make_async_copy(ob1, out.at[pl.ds(0, KA)], sw1).wait()


@functools.cache
def _sc_scatter_kernel():
    return pl.kernel(
        _sc_scatter_body,
        out_type=jax.ShapeDtypeStruct((2 * NP, 128), jnp.float32),
        mesh=_sc_mesh(),
        scratch_types=[
            pltpu.VMEM((EPT,), jnp.int32),
            pltpu.VMEM((EPT,), jnp.int32),
            pltpu.VMEM((2, KS), jnp.float32),
            pltpu.VMEM((KS, 128), jnp.float32),
            pltpu.VMEM((KS, 128), jnp.float32),
            pltpu.VMEM((ZB, 128), jnp.float32),
            pltpu.VMEM_SHARED((NP, 128), jnp.float32),
            pltpu.SemaphoreType.DMA,
            pltpu.SemaphoreType.DMA,
            pltpu.SemaphoreType.DMA,
            pltpu.SemaphoreType.DMA,
        ],
    )


def _sc_scatter_body(table2, src, dst, w, out, si, di, wv, rows0, rows1,
                     zbuf, agg, sg0, sg1, sv0, sv1):
    cid = lax.axis_index("c")
    sid = lax.axis_index("s")
    rowsb = (rows0, rows1)
    sgb = (sg0, sg1)
    svb = (sv0, sv1)
    base = sid * EPT
    col_off = cid * NP
    nch = EPT // KS
    rows_per_tile = NP // NTILES

    pltpu.sync_copy(src.at[pl.ds(base, EPT)], si)
    pltpu.sync_copy(dst.at[pl.ds(base, EPT)], di)

    def adj(g, carry2):
        sl = pl.ds(g * 16, 16)
        si[sl] = si[sl] + col_off
        return carry2

    lax.fori_loop(0, EPT // 16, adj, 0)

    def zrow(r, carry):
        for g in range(8):
            zbuf[r, pl.ds(g * 16, 16)] = jnp.zeros((16,), jnp.float32)
        return carry

    lax.fori_loop(0, ZB, zrow, 0)
    for z in range(rows_per_tile // ZB):
        pltpu.sync_copy(zbuf, agg.at[pl.ds(sid * rows_per_tile + z * ZB, ZB)])
    plsc.subcore_barrier()

    def issue(c, b):
        pltpu.async_copy(table2.at[si.at[pl.ds(c * KS, KS)]], rowsb[b], sgb[b])

    pltpu.sync_copy(w.at[pl.ds(base, KS)], wv.at[0])
    pltpu.sync_copy(w.at[pl.ds(base + KS, KS)], wv.at[1])
    issue(0, 0)
    issue(1, 1)

    def pair(cc, carry):
        for b in range(2):
            c = cc * 2 + b
            pltpu.make_async_copy(table2.at[si.at[pl.ds(0, KS)]],
                                  rowsb[b], sgb[b]).wait()

            @pl.when(cc > 0)
            def _():
                pltpu.make_async_copy(w.at[pl.ds(0, KS)], wv.at[b],
                                      svb[b]).wait()

            def scale(q, carry2):
                w16 = wv[b, pl.ds(q * 16, 16)]
                for j in range(16):
                    r = q * 16 + j
                    ws = w16[j]
                    for g in range(8):
                        sl = pl.ds(g * 16, 16)
                        rowsb[b][r, sl] = rowsb[b][r, sl] * ws
                return carry2

            lax.fori_loop(0, KS // 16, scale, 0)
            pltpu.sync_copy(rowsb[b], agg.at[di.at[pl.ds(c * KS, KS)]],
                            add=True)

            @pl.when(cc < nch // 2 - 1)
            def _():
                pltpu.async_copy(w.at[pl.ds(base + (c + 2) * KS, KS)],
                                 wv.at[b], svb[b])
                issue(c + 2, b)
        return carry

    lax.fori_loop(0, nch // 2, pair, 0)
    plsc.subcore_barrier()
    pltpu.sync_copy(
        agg.at[pl.ds(sid * rows_per_tile, rows_per_tile)],
        out.at[pl.ds(col_off + sid * rows_per_tile, rows_per_tile)],
    )


def kernel(x, edge_index, edge_weight, g_size, W_emb, b_emb,
           wc1_W1, wc1_b1, wc1_W2, wc1_b2, gc1_W, gc1_b,
           wc2_W1, wc2_b1, wc2_W2, wc2_b2, gc2_W, gc2_b,
           wcl_W1, wcl_b1, wcl_W2, wcl_b2, fc_W, fc_b):
    src = jnp.pad(edge_index[0], (0, EP - N_EDGES))
    dst = jnp.pad(edge_index[1], (0, EP - N_EDGES))
    edge_weight = jnp.pad(edge_weight, (0, EP - N_EDGES))

    x_p = jnp.pad(x, ((0, NP - N_NODES), (0, 0)))
    h0 = _tc_matmul([x_p], [W_emb], b_emb, relu=True)

    def chunks(h):
        return jnp.concatenate([h[:, :128], h[:, 128:]], axis=0)

    d0 = _sc_absdiff_kernel()(h0, src, dst)
    w1 = _tc_edge_mlp([d0], [wc1_W1], wc1_b1, wc1_W2[:, 0], wc1_b2,
                      edge_weight)
    agg1 = _sc_scatter_kernel()(chunks(h0), src, dst, w1)
    hn1 = _tc_matmul(
        [h0, agg1[:NP], agg1[NP:]],
        [gc1_W[0:256], gc1_W[256:384], gc1_W[384:512]],
        gc1_b, relu=True)

    d1 = _sc_absdiff_kernel()(hn1, src, dst)
    w2 = _tc_edge_mlp([d0, d1], [wc2_W1[0:256], wc2_W1[256:512]],
                      wc2_b1, wc2_W2[:, 0], wc2_b2, edge_weight)
    agg2a = _sc_scatter_kernel()(chunks(h0), src, dst, w2)
    agg2b = _sc_scatter_kernel()(chunks(hn1), src, dst, w2)
    hn2 = _tc_matmul(
        [h0, hn1, agg2a[:NP], agg2a[NP:], agg2b[:NP], agg2b[NP:]],
        [gc2_W[0:256], gc2_W[256:512], gc2_W[512:640], gc2_W[640:768],
         gc2_W[768:896], gc2_W[896:1024]],
        gc2_b, relu=True)

    d2 = _sc_absdiff_kernel()(hn2, src, dst)
    wl = _tc_edge_mlp([d0, d1, d2],
                      [wcl_W1[0:256], wcl_W1[256:512], wcl_W1[512:768]],
                      wcl_b1, wcl_W2[:, 0], wcl_b2, edge_weight)[:N_EDGES]
    out = _tc_matmul([h0, hn1, hn2],
                     [fc_W[0:256], fc_W[256:512], fc_W[512:768]],
                     fc_b, relu=False)[:N_NODES]
    return (out, wl, g_size)
